# baseline (speedup 1.0000x reference)
import functools

import jax
import jax.numpy as jnp
from jax import lax
from jax.experimental import pallas as pl
from jax.experimental.pallas import tpu as pltpu
from jax.experimental.pallas import tpu_sc as plsc

B, R, N, K = 8, 32, 128, 64
ROWS = B * R
NC, NS, L = 2, 16, 16
NW = NC * NS
ROWS_PER_W = ROWS // NW
XW_PER_W = ROWS_PER_W * N // 2
ELEMS_PER_W = ROWS_PER_W * K
OW_PER_W = ELEMS_PER_W // 2
WVECS = OW_PER_W // L

_mesh = plsc.VectorSubcoreMesh(core_axis_name="c", subcore_axis_name="s")


@functools.partial(
    pl.kernel,
    mesh=_mesh,
    out_type=jax.ShapeDtypeStruct((ROWS * K // 2,), jnp.int32),
    scratch_types=[
        pltpu.VMEM((XW_PER_W,), jnp.int32),
        pltpu.VMEM((ELEMS_PER_W,), jnp.int32),
        pltpu.VMEM((OW_PER_W,), jnp.int32),
    ],
    compiler_params=pltpu.CompilerParams(needs_layout_passes=False),
)
def _gather_sc(xw_hbm, idx_hbm, ow_hbm, xw_v, idx_v, ow_v):
    wid = lax.axis_index("s") * NC + lax.axis_index("c")
    pltpu.sync_copy(xw_hbm.at[pl.ds(wid * XW_PER_W, XW_PER_W)], xw_v)
    pltpu.sync_copy(idx_hbm.at[pl.ds(wid * ELEMS_PER_W, ELEMS_PER_W)], idx_v)
    pos2 = lax.iota(jnp.int32, L) * 2
    for v in range(WVECS):
        pe = pos2 + v * 2 * L
        po = pe + 1
        row_base = (v // 2) * N
        ee = plsc.load_gather(idx_v, [pe]) + row_base
        eo = plsc.load_gather(idx_v, [po]) + row_base
        ve = plsc.load_gather(xw_v, [lax.shift_right_logical(ee, 1)])
        vo = plsc.load_gather(xw_v, [lax.shift_right_logical(eo, 1)])
        se = lax.shift_right_logical(ve, lax.shift_left(ee & 1, 4)) & 0xFFFF
        so = lax.shift_right_logical(vo, lax.shift_left(eo & 1, 4)) & 0xFFFF
        ow_v[pl.ds(v * L, L)] = se | lax.shift_left(so, 16)
    pltpu.sync_copy(ow_v, ow_hbm.at[pl.ds(wid * OW_PER_W, OW_PER_W)])


def kernel(x, index, dim):
    del dim
    xw = lax.bitcast_convert_type(x.reshape(-1, 2), jnp.int32)
    idxf = index.astype(jnp.int32).reshape(-1)
    ow = _gather_sc(xw, idxf)
    out = lax.bitcast_convert_type(ow, x.dtype).reshape(index.shape)
    return out


# speedup vs baseline: 1.5985x; 1.5985x over previous
import functools

import jax
import jax.numpy as jnp
from jax import lax
from jax.experimental import pallas as pl
from jax.experimental.pallas import tpu as pltpu
from jax.experimental.pallas import tpu_sc as plsc

B, R, N, K = 8, 32, 128, 64
ROWS = B * R
NC, NS, L = 2, 16, 16
NW = NC * NS
RPW = ROWS // NW
EPW = RPW * K
VPR = K // L
CH = 128

_mesh = plsc.VectorSubcoreMesh(core_axis_name="c", subcore_axis_name="s")


@functools.partial(
    pl.kernel,
    mesh=_mesh,
    out_type=jax.ShapeDtypeStruct((ROWS * K,), jnp.float32),
    scratch_types=[
        pltpu.VMEM((EPW,), jnp.int32),
        pltpu.VMEM((EPW,), jnp.float32),
        pltpu.SemaphoreType.DMA,
    ],
    compiler_params=pltpu.CompilerParams(needs_layout_passes=False),
)
def _gather_sc(x_hbm, idx_hbm, out_hbm, idx_v, o_v, sem):
    wid = lax.axis_index("s") * NC + lax.axis_index("c")
    base = wid * EPW
    pltpu.sync_copy(idx_hbm.at[pl.ds(base, EPW)], idx_v)
    for i in range(EPW // L):
        row_base = (wid * RPW + i // VPR) * N
        idx_v[pl.ds(i * L, L)] = idx_v[pl.ds(i * L, L)] + row_base
    cps = [
        pltpu.async_copy(
            x_hbm.at[idx_v.at[pl.ds(j * CH, CH)]],
            o_v.at[pl.ds(j * CH, CH)],
            sem,
        )
        for j in range(EPW // CH)
    ]
    for cp in cps:
        cp.wait()
    pltpu.sync_copy(o_v, out_hbm.at[pl.ds(base, EPW)])


def kernel(x, index, dim):
    del dim
    xf = x.reshape(-1).astype(jnp.float32)
    idxf = index.astype(jnp.int32).reshape(-1)
    out = _gather_sc(xf, idxf)
    return out.reshape(index.shape).astype(x.dtype)


# speedup vs baseline: 1.6181x; 1.0122x over previous
import functools

import jax
import jax.numpy as jnp
from jax import lax
from jax.experimental import pallas as pl
from jax.experimental.pallas import tpu as pltpu
from jax.experimental.pallas import tpu_sc as plsc

B, R, N, K = 8, 32, 128, 64
ROWS = B * R
NC, NS, L = 2, 16, 16
NW = NC * NS
RPW = ROWS // NW
EPW = RPW * K
VECS = EPW // L
VPR = K // L

_mesh = plsc.VectorSubcoreMesh(core_axis_name="c", subcore_axis_name="s")


@functools.partial(
    pl.kernel,
    mesh=_mesh,
    out_type=jax.ShapeDtypeStruct((ROWS * K,), jnp.float32),
    scratch_types=[
        pltpu.VMEM((RPW * N,), jnp.float32),
        pltpu.VMEM((EPW,), jnp.int32),
        pltpu.VMEM((EPW,), jnp.float32),
        pltpu.SemaphoreType.DMA,
        pltpu.SemaphoreType.DMA,
    ],
    compiler_params=pltpu.CompilerParams(needs_layout_passes=False),
)
def _gather_sc(x_hbm, idx_hbm, out_hbm, x_v, idx_v, o_v, sem_x, sem_i):
    wid = lax.axis_index("s") * NC + lax.axis_index("c")
    e_base = wid * EPW
    cp_x = pltpu.async_copy(x_hbm.at[pl.ds(wid * RPW * N, RPW * N)], x_v, sem_x)
    cp_i = pltpu.async_copy(idx_hbm.at[pl.ds(e_base, EPW)], idx_v, sem_i)
    cp_i.wait()
    cp_x.wait()
    for i in range(VECS):
        idx = idx_v[pl.ds(i * L, L)] + (i // VPR) * N
        o_v[pl.ds(i * L, L)] = plsc.load_gather(x_v, [idx])
    pltpu.sync_copy(o_v, out_hbm.at[pl.ds(e_base, EPW)])


def kernel(x, index, dim):
    del dim
    xf = x.reshape(-1).astype(jnp.float32)
    idxf = index.astype(jnp.int32).reshape(-1)
    out = _gather_sc(xf, idxf)
    return out.reshape(index.shape).astype(x.dtype)


# speedup vs baseline: 1.7646x; 1.0906x over previous
import functools

import jax
import jax.numpy as jnp
from jax import lax
from jax.experimental import pallas as pl
from jax.experimental.pallas import tpu as pltpu
from jax.experimental.pallas import tpu_sc as plsc

B, R, N, K = 8, 32, 128, 64
ROWS = B * R
NC, NS, L = 1, 16, 16
NW = NC * NS
RPW = ROWS // NW
EPW = RPW * K
VECS = EPW // L
VPR = K // L

_mesh = plsc.VectorSubcoreMesh(
    core_axis_name="c", subcore_axis_name="s", num_cores=1
)


@functools.partial(
    pl.kernel,
    mesh=_mesh,
    out_type=jax.ShapeDtypeStruct((ROWS * K,), jnp.float32),
    scratch_types=[
        pltpu.VMEM((RPW * N,), jnp.float32),
        pltpu.VMEM((EPW,), jnp.int32),
        pltpu.VMEM((EPW,), jnp.float32),
        pltpu.SemaphoreType.DMA,
        pltpu.SemaphoreType.DMA,
    ],
    compiler_params=pltpu.CompilerParams(needs_layout_passes=False),
)
def _gather_sc(x_hbm, idx_hbm, out_hbm, x_v, idx_v, o_v, sem_x, sem_i):
    wid = lax.axis_index("s") * NC + lax.axis_index("c")
    e_base = wid * EPW
    cp_x = pltpu.async_copy(x_hbm.at[pl.ds(wid * RPW * N, RPW * N)], x_v, sem_x)
    cp_i = pltpu.async_copy(idx_hbm.at[pl.ds(e_base, EPW)], idx_v, sem_i)
    cp_i.wait()
    cp_x.wait()
    for i in range(VECS):
        idx = idx_v[pl.ds(i * L, L)] + (i // VPR) * N
        o_v[pl.ds(i * L, L)] = plsc.load_gather(x_v, [idx])
    pltpu.sync_copy(o_v, out_hbm.at[pl.ds(e_base, EPW)])


def kernel(x, index, dim):
    del dim
    xf = x.reshape(-1).astype(jnp.float32)
    idxf = index.astype(jnp.int32).reshape(-1)
    out = _gather_sc(xf, idxf)
    return out.reshape(index.shape).astype(x.dtype)


# speedup vs baseline: 1.7729x; 1.0047x over previous
import functools

import jax
import jax.numpy as jnp
from jax import lax
from jax.experimental import pallas as pl
from jax.experimental.pallas import tpu as pltpu
from jax.experimental.pallas import tpu_sc as plsc

B, R, N, K = 8, 32, 128, 64
ROWS = B * R
NS, L = 16, 16
NW = NS
RPW = ROWS // NW
WPB = R // RPW
EPW = RPW * K
VECS = EPW // L
VPR = K // L

_mesh = plsc.VectorSubcoreMesh(
    core_axis_name="c", subcore_axis_name="s", num_cores=1
)


@functools.partial(
    pl.kernel,
    mesh=_mesh,
    out_type=jax.ShapeDtypeStruct((B, R, K), jnp.float32),
    scratch_types=[
        pltpu.VMEM((RPW * N,), jnp.float32),
        pltpu.VMEM((EPW,), jnp.int32),
        pltpu.VMEM((RPW, K), jnp.float32),
        pltpu.SemaphoreType.DMA,
        pltpu.SemaphoreType.DMA,
    ],
    compiler_params=pltpu.CompilerParams(needs_layout_passes=False),
)
def _gather_sc(x_hbm, idx_hbm, out_hbm, x_v, idx_v, o_v, sem_x, sem_i):
    wid = lax.axis_index("s")
    cp_x = pltpu.async_copy(x_hbm.at[pl.ds(wid * RPW * N, RPW * N)], x_v, sem_x)
    cp_i = pltpu.async_copy(idx_hbm.at[pl.ds(wid * EPW, EPW)], idx_v, sem_i)
    cp_i.wait()
    cp_x.wait()
    for i in range(VECS):
        r = i // VPR
        idx = idx_v[pl.ds(i * L, L)] + r * N
        o_v[r, pl.ds((i % VPR) * L, L)] = plsc.load_gather(x_v, [idx])
    pltpu.sync_copy(
        o_v, out_hbm.at[wid // WPB, pl.ds((wid % WPB) * RPW, RPW), :]
    )


def kernel(x, index, dim):
    del dim
    xf = x.reshape(-1).astype(jnp.float32)
    idxf = index.astype(jnp.int32).reshape(-1)
    out = _gather_sc(xf, idxf)
    return out.astype(x.dtype)


# speedup vs baseline: 1.7826x; 1.0054x over previous
import functools

import jax
import jax.numpy as jnp
from jax import lax
from jax.experimental import pallas as pl
from jax.experimental.pallas import tpu as pltpu
from jax.experimental.pallas import tpu_sc as plsc

B, R, N, K = 8, 32, 128, 64
ROWS = B * R
NS, L = 16, 16
NW = NS
RPW = ROWS // NW
WPB = R // RPW
EPW = RPW * K
VECS = EPW // L
VPR = K // L

_mesh = plsc.VectorSubcoreMesh(
    core_axis_name="c", subcore_axis_name="s", num_cores=1
)


@functools.partial(
    pl.kernel,
    mesh=_mesh,
    out_type=jax.ShapeDtypeStruct((B, R, K), jnp.float32),
    scratch_types=[
        pltpu.VMEM((RPW * N,), jnp.float32),
        pltpu.VMEM((EPW,), jnp.int32),
        pltpu.VMEM((RPW, K), jnp.float32),
        pltpu.SemaphoreType.DMA,
        pltpu.SemaphoreType.DMA,
        pltpu.SemaphoreType.DMA,
    ],
    compiler_params=pltpu.CompilerParams(needs_layout_passes=False),
)
def _gather_sc(x_hbm, idx_hbm, out_hbm, x_v, idx_v, o_v, sem_x, sem_i, sem_o):
    wid = lax.axis_index("s")
    cp_x = pltpu.async_copy(x_hbm.at[pl.ds(wid * RPW * N, RPW * N)], x_v, sem_x)
    cp_i = pltpu.async_copy(idx_hbm.at[pl.ds(wid * EPW, EPW)], idx_v, sem_i)
    cp_i.wait()
    cp_x.wait()
    b, r0 = wid // WPB, (wid % WPB) * RPW
    half = RPW // 2
    cps = []
    for h in range(2):
        for i in range(h * VECS // 2, (h + 1) * VECS // 2):
            r = i // VPR
            idx = idx_v[pl.ds(i * L, L)] + r * N
            o_v[r, pl.ds((i % VPR) * L, L)] = plsc.load_gather(x_v, [idx])
        cps.append(
            pltpu.async_copy(
                o_v.at[pl.ds(h * half, half), :],
                out_hbm.at[b, pl.ds(r0 + h * half, half), :],
                sem_o,
            )
        )
    for cp in cps:
        cp.wait()


def kernel(x, index, dim):
    del dim
    xf = x.reshape(-1).astype(jnp.float32)
    idxf = index.astype(jnp.int32).reshape(-1)
    out = _gather_sc(xf, idxf)
    return out.astype(x.dtype)
